# Initial kernel scaffold; baseline (speedup 1.0000x reference)
import functools

import jax
import jax.numpy as jnp
from jax import lax
from jax.experimental import pallas as pl
from jax.experimental.pallas import tpu as pltpu
from jax.experimental.pallas import tpu_sc as plsc

_NX = 128
_C = 64
_LAMDIFF = 0.01
_DELTA = 0.001
_N = 320000

_NC = 2
_NS = 16
_NW = _NC * _NS
_G = 128
_NGRP = _N // _G
_ROUNDS = -(-_NGRP // _NW)


def _sc_segment_sums(x_hbm, y_hbm, zsum_hbm, zcnt_hbm, ones_hbm,
                     sums_out, cnts_out,
                     xbuf, ybuf, ones_v, ssum, scnt):
    c = lax.axis_index("c")
    s = lax.axis_index("s")
    wid = c * _NS + s

    @pl.when(s == 0)
    def _():
        pltpu.sync_copy(zsum_hbm, ssum)
        pltpu.sync_copy(zcnt_hbm, scnt)

    pltpu.sync_copy(ones_hbm, ones_v)
    plsc.subcore_barrier()

    def round_body(i, carry):
        g = wid + i * _NW

        @pl.when(g < _NGRP)
        def _():
            pltpu.sync_copy(x_hbm.at[pl.ds(g * _G, _G)], xbuf)
            pltpu.sync_copy(y_hbm.at[g], ybuf)
            pltpu.sync_copy(xbuf, ssum.at[ybuf], add=True)
            pltpu.sync_copy(ones_v, scnt.at[ybuf], add=True)

        return carry

    lax.fori_loop(0, _ROUNDS, round_body, 0)
    plsc.subcore_barrier()

    @pl.when(s == 0)
    def _():
        pltpu.sync_copy(ssum, sums_out.at[c])
        pltpu.sync_copy(scnt, cnts_out.at[c])


@jax.jit
def _sc_call(x, y2, zsum, zcnt, ones):
    mesh = plsc.VectorSubcoreMesh(core_axis_name="c", subcore_axis_name="s",
                                  num_cores=_NC, num_subcores=_NS)
    return pl.kernel(
        _sc_segment_sums,
        out_type=(
            jax.ShapeDtypeStruct((_NC, _C, _NX), jnp.float32),
            jax.ShapeDtypeStruct((_NC, _C, 16), jnp.float32),
        ),
        mesh=mesh,
        scratch_types=[
            pltpu.VMEM((_G, _NX), jnp.float32),
            pltpu.VMEM((_G,), jnp.int32),
            pltpu.VMEM((_G, 16), jnp.float32),
            pltpu.VMEM_SHARED((_C, _NX), jnp.float32),
            pltpu.VMEM_SHARED((_C, 16), jnp.float32),
        ],
    )(x, y2, zsum, zcnt, ones)


def _tc_epilogue(sums_ref, cnts_ref, m1_ref, m2_ref, l1_ref, l2_ref, o_ref):
    sums = sums_ref[0] + sums_ref[1]
    counts = cnts_ref[0][:, 0:1] + cnts_ref[1][:, 0:1]
    xbar = sums / counts
    m1 = m1_ref[...]
    m2 = m2_ref[...]
    dd1 = xbar - m1
    dd2 = xbar - m2
    d1 = jnp.sum(dd1 * dd1, axis=1, keepdims=True)
    d2 = jnp.sum(dd2 * dd2, axis=1, keepdims=True)
    cond = d1 < d2
    l1 = l1_ref[...]
    l2 = l2_ref[...]
    l1a = jnp.clip(l1 - _DELTA, 0.0, 1.0)
    l2a = l1a + _LAMDIFF
    l2b = jnp.clip(l2 + _DELTA, 0.0, 1.0)
    l1b = l2b - _LAMDIFF
    l1n = jnp.where(cond, l1a, l1b)
    l2n = jnp.where(cond, l2a, l2b)
    m1n = (1.0 - l1n) * xbar + l1n * m1
    m2n = (1.0 - l2n) * xbar + l2n * m2
    me = 0.5 * (m1n + m2n)
    colsum = jnp.sum(me, axis=0, keepdims=True)
    val = _C * jnp.sum(me * me) - jnp.sum(colsum * colsum)
    o_ref[0, 0] = jnp.sqrt(jnp.maximum(val, 0.0))


def kernel(x, y, m1, m2, lam1, lam2):
    y2 = y.reshape(_NGRP, _G)
    zsum = jnp.zeros((_C, _NX), jnp.float32)
    zcnt = jnp.zeros((_C, 16), jnp.float32)
    ones = jnp.ones((_G, 16), jnp.float32)
    sums, cnts = _sc_call(x, y2, zsum, zcnt, ones)
    out = pl.pallas_call(
        _tc_epilogue,
        out_shape=jax.ShapeDtypeStruct((1, 1), jnp.float32),
    )(sums, cnts, m1, m2, lam1.reshape(_C, 1), lam2.reshape(_C, 1))
    return out[0, 0]


# speedup vs baseline: 5.9970x; 5.9970x over previous
import functools

import jax
import jax.numpy as jnp
from jax import lax
from jax.experimental import pallas as pl
from jax.experimental.pallas import tpu as pltpu
from jax.experimental.pallas import tpu_sc as plsc

_NX = 128
_C = 64
_LAMDIFF = 0.01
_DELTA = 0.001
_N = 320000

_NC = 2
_NS = 16
_NW = _NC * _NS
_G = 128
_NGRP = _N // _G
_ROUNDS = -(-_NGRP // _NW)


def _sc_segment_sums(x_hbm, y_hbm, zsum_hbm, zcnt_hbm, ones_hbm,
                     sums_out, cnts_out,
                     xbuf, ybuf, ones_v, ssum, scnt):
    c = lax.axis_index("c")
    s = lax.axis_index("s")
    wid = c * _NS + s

    @pl.when(s == 0)
    def _():
        pltpu.sync_copy(zsum_hbm, ssum)
        pltpu.sync_copy(zcnt_hbm, scnt)

    pltpu.sync_copy(ones_hbm, ones_v)
    plsc.subcore_barrier()

    def round_body(i, carry):
        g = wid + i * _NW

        @pl.when(g < _NGRP)
        def _():
            pltpu.sync_copy(x_hbm.at[pl.ds(g * _G, _G)], xbuf)
            pltpu.sync_copy(y_hbm.at[g], ybuf)
            pltpu.sync_copy(xbuf, ssum.at[ybuf], add=True)
            pltpu.sync_copy(ones_v, scnt.at[ybuf], add=True)

        return carry

    lax.fori_loop(0, _ROUNDS, round_body, 0)
    plsc.subcore_barrier()

    @pl.when(s == 0)
    def _():
        pltpu.sync_copy(ssum, sums_out.at[c])
        pltpu.sync_copy(scnt, cnts_out.at[c])


@jax.jit
def _sc_call(x, y2, zsum, zcnt, ones):
    mesh = plsc.VectorSubcoreMesh(core_axis_name="c", subcore_axis_name="s",
                                  num_cores=_NC, num_subcores=_NS)
    return pl.kernel(
        _sc_segment_sums,
        out_type=(
            jax.ShapeDtypeStruct((_NC, _C, _NX), jnp.float32),
            jax.ShapeDtypeStruct((_NC, _C, 16), jnp.float32),
        ),
        mesh=mesh,
        scratch_types=[
            pltpu.VMEM((_G, _NX), jnp.float32),
            pltpu.VMEM((_G,), jnp.int32),
            pltpu.VMEM((_G, 16), jnp.float32),
            pltpu.VMEM_SHARED((_C, _NX), jnp.float32),
            pltpu.VMEM_SHARED((_C, 16), jnp.float32),
        ],
    )(x, y2, zsum, zcnt, ones)


def _tc_epilogue(sums_ref, cnts_ref, m1_ref, m2_ref, l1_ref, l2_ref, o_ref):
    sums = sums_ref[0] + sums_ref[1]
    counts = cnts_ref[0][:, 0:1] + cnts_ref[1][:, 0:1]
    xbar = sums / counts
    m1 = m1_ref[...]
    m2 = m2_ref[...]
    dd1 = xbar - m1
    dd2 = xbar - m2
    d1 = jnp.sum(dd1 * dd1, axis=1, keepdims=True)
    d2 = jnp.sum(dd2 * dd2, axis=1, keepdims=True)
    cond = d1 < d2
    l1 = l1_ref[...]
    l2 = l2_ref[...]
    l1a = jnp.clip(l1 - _DELTA, 0.0, 1.0)
    l2a = l1a + _LAMDIFF
    l2b = jnp.clip(l2 + _DELTA, 0.0, 1.0)
    l1b = l2b - _LAMDIFF
    l1n = jnp.where(cond, l1a, l1b)
    l2n = jnp.where(cond, l2a, l2b)
    m1n = (1.0 - l1n) * xbar + l1n * m1
    m2n = (1.0 - l2n) * xbar + l2n * m2
    me = 0.5 * (m1n + m2n)
    colsum = jnp.sum(me, axis=0, keepdims=True)
    val = _C * jnp.sum(me * me) - jnp.sum(colsum * colsum)
    o_ref[...] = jnp.sqrt(jnp.maximum(val, 0.0)).reshape(1, 1)


def kernel(x, y, m1, m2, lam1, lam2):
    y2 = y.reshape(_NGRP, _G)
    zsum = jnp.zeros((_C, _NX), jnp.float32)
    zcnt = jnp.zeros((_C, 16), jnp.float32)
    ones = jnp.ones((_G, 16), jnp.float32)
    sums, cnts = _sc_call(x, y2, zsum, zcnt, ones)
    out = pl.pallas_call(
        _tc_epilogue,
        out_shape=jax.ShapeDtypeStruct((1, 1), jnp.float32),
    )(sums, cnts, m1, m2, lam1.reshape(_C, 1), lam2.reshape(_C, 1))
    return out[0, 0]


# speedup vs baseline: 11.4261x; 1.9053x over previous
import functools

import jax
import jax.numpy as jnp
from jax import lax
from jax.experimental import pallas as pl
from jax.experimental.pallas import tpu as pltpu
from jax.experimental.pallas import tpu_sc as plsc

_NX = 128
_C = 64
_LAMDIFF = 0.01
_DELTA = 0.001
_N = 320000

_NC = 2
_NS = 16
_NW = _NC * _NS
_G = 128
_NGRP = _N // _G
_ROUNDS = -(-_NGRP // _NW)


def _sc_segment_sums(x_hbm, y_hbm, zsum_hbm, zcnt_hbm, ones_hbm,
                     sums_out, cnts_out,
                     xbuf0, xbuf1, ybuf0, ybuf1, sem0, sem1,
                     ones_v, ssum, scnt):
    c = lax.axis_index("c")
    s = lax.axis_index("s")
    wid = c * _NS + s
    xbufs = (xbuf0, xbuf1)
    ybufs = (ybuf0, ybuf1)
    sems = (sem0, sem1)

    @pl.when(s == 0)
    def _():
        pltpu.sync_copy(zsum_hbm, ssum)
        pltpu.sync_copy(zcnt_hbm, scnt)

    pltpu.sync_copy(ones_hbm, ones_v)
    plsc.subcore_barrier()

    def start_load(g, b):
        @pl.when(g < _NGRP)
        def _():
            pltpu.async_copy(x_hbm.at[pl.ds(g * _G, _G)], xbufs[b], sems[b])
            pltpu.async_copy(y_hbm.at[g], ybufs[b], sems[b])

    def wait_load(g, b):
        pltpu.make_async_copy(x_hbm.at[pl.ds(g * _G, _G)], xbufs[b],
                              sems[b]).wait()
        pltpu.make_async_copy(y_hbm.at[g], ybufs[b], sems[b]).wait()

    def consume(g, b):
        @pl.when(g < _NGRP)
        def _():
            wait_load(g, b)
            pltpu.sync_copy(xbufs[b], ssum.at[ybufs[b]], add=True)
            pltpu.sync_copy(ones_v, scnt.at[ybufs[b]], add=True)

    start_load(wid, 0)
    start_load(wid + _NW, 1)

    def round_body(k, carry):
        g0 = wid + (2 * k) * _NW
        g1 = wid + (2 * k + 1) * _NW
        consume(g0, 0)
        start_load(g0 + 2 * _NW, 0)
        consume(g1, 1)
        start_load(g1 + 2 * _NW, 1)
        return carry

    lax.fori_loop(0, -(-_ROUNDS // 2), round_body, 0)
    plsc.subcore_barrier()

    @pl.when(s == 0)
    def _():
        pltpu.sync_copy(ssum, sums_out.at[c])
        pltpu.sync_copy(scnt, cnts_out.at[c])


@jax.jit
def _sc_call(x, y2, zsum, zcnt, ones):
    mesh = plsc.VectorSubcoreMesh(core_axis_name="c", subcore_axis_name="s",
                                  num_cores=_NC, num_subcores=_NS)
    return pl.kernel(
        _sc_segment_sums,
        out_type=(
            jax.ShapeDtypeStruct((_NC, _C, _NX), jnp.float32),
            jax.ShapeDtypeStruct((_NC, _C, 16), jnp.float32),
        ),
        mesh=mesh,
        scratch_types=[
            pltpu.VMEM((_G, _NX), jnp.float32),
            pltpu.VMEM((_G, _NX), jnp.float32),
            pltpu.VMEM((_G,), jnp.int32),
            pltpu.VMEM((_G,), jnp.int32),
            pltpu.SemaphoreType.DMA,
            pltpu.SemaphoreType.DMA,
            pltpu.VMEM((_G, 16), jnp.float32),
            pltpu.VMEM_SHARED((_C, _NX), jnp.float32),
            pltpu.VMEM_SHARED((_C, 16), jnp.float32),
        ],
    )(x, y2, zsum, zcnt, ones)


def _tc_epilogue(sums_ref, cnts_ref, m1_ref, m2_ref, l1_ref, l2_ref, o_ref):
    sums = sums_ref[0] + sums_ref[1]
    counts = cnts_ref[0][:, 0:1] + cnts_ref[1][:, 0:1]
    xbar = sums / counts
    m1 = m1_ref[...]
    m2 = m2_ref[...]
    dd1 = xbar - m1
    dd2 = xbar - m2
    d1 = jnp.sum(dd1 * dd1, axis=1, keepdims=True)
    d2 = jnp.sum(dd2 * dd2, axis=1, keepdims=True)
    cond = d1 < d2
    l1 = l1_ref[...]
    l2 = l2_ref[...]
    l1a = jnp.clip(l1 - _DELTA, 0.0, 1.0)
    l2a = l1a + _LAMDIFF
    l2b = jnp.clip(l2 + _DELTA, 0.0, 1.0)
    l1b = l2b - _LAMDIFF
    l1n = jnp.where(cond, l1a, l1b)
    l2n = jnp.where(cond, l2a, l2b)
    m1n = (1.0 - l1n) * xbar + l1n * m1
    m2n = (1.0 - l2n) * xbar + l2n * m2
    me = 0.5 * (m1n + m2n)
    colsum = jnp.sum(me, axis=0, keepdims=True)
    val = _C * jnp.sum(me * me) - jnp.sum(colsum * colsum)
    o_ref[...] = jnp.sqrt(jnp.maximum(val, 0.0)).reshape(1, 1)


def kernel(x, y, m1, m2, lam1, lam2):
    y2 = y.reshape(_NGRP, _G)
    zsum = jnp.zeros((_C, _NX), jnp.float32)
    zcnt = jnp.zeros((_C, 16), jnp.float32)
    ones = jnp.ones((_G, 16), jnp.float32)
    sums, cnts = _sc_call(x, y2, zsum, zcnt, ones)
    out = pl.pallas_call(
        _tc_epilogue,
        out_shape=jax.ShapeDtypeStruct((1, 1), jnp.float32),
    )(sums, cnts, m1, m2, lam1.reshape(_C, 1), lam2.reshape(_C, 1))
    return out[0, 0]
